# baseline (speedup 1.0000x reference)
import functools

import jax
import jax.numpy as jnp
from jax import lax
from jax.experimental import pallas as pl
from jax.experimental.pallas import tpu as pltpu
from jax.experimental.pallas import tpu_sc as plsc

BATCH = 4096
SEQ_LEN = 200
D_MODEL = 64
VOCAB_SZ = 1000000
PAD_W = 128
LANES = 16

NUM_CORES = 2
NUM_SUBCORES = 16
NUM_WORKERS = NUM_CORES * NUM_SUBCORES
SEQ_PER_WORKER = BATCH // NUM_WORKERS
CH = (104, 96)
SOFF = (0, 104)
NCH = 2 * SEQ_PER_WORKER

_mesh = plsc.VectorSubcoreMesh(core_axis_name="c", subcore_axis_name="s")
_params = pltpu.CompilerParams(use_tc_tiling_on_sc=True)


@functools.partial(
    pl.kernel,
    mesh=_mesh,
    compiler_params=_params,
    out_type=jax.ShapeDtypeStruct((BATCH, SEQ_LEN, D_MODEL), jnp.float32),
    scratch_types=[
        pltpu.VMEM((256,), jnp.int32),
        pltpu.VMEM((256,), jnp.int32),
        pltpu.VMEM((SEQ_LEN // 2, PAD_W), jnp.float32),
        pltpu.VMEM((CH[0], PAD_W), jnp.float32),
        pltpu.VMEM((CH[1], PAD_W), jnp.float32),
        pltpu.VMEM((CH[0], D_MODEL), jnp.float32),
        pltpu.VMEM((CH[1], D_MODEL), jnp.float32),
        pltpu.SemaphoreType.DMA((2,)),
        pltpu.SemaphoreType.DMA((2,)),
        pltpu.SemaphoreType.DMA((2,)),
    ],
)
def _emb_kernel(
    x_hbm, pair_hbm, pos_hbm, out_hbm,
    ridx, jdx, pos_v, buf0, buf1, obuf0, obuf1, isem, gsem, ssem,
):
    cid = lax.axis_index("c")
    sid = lax.axis_index("s")
    wid = sid * NUM_CORES + cid
    seq_base = wid * SEQ_PER_WORKER

    bufs = (buf0, buf1)
    obufs = (obuf0, obuf1)

    pltpu.sync_copy(pos_hbm, pos_v)

    def x_off(c, b):
        seq = lax.shift_right_logical(c, 1)
        return (seq_base + seq) * SEQ_LEN + SOFF[b]

    def start_idload(c, b):
        pltpu.async_copy(
            x_hbm.at[pl.ds(x_off(c, b), CH[b])],
            ridx.at[pl.ds(b * 128, CH[b])],
            isem.at[b],
        )

    def wait_idload(b):
        pltpu.make_async_copy(
            x_hbm.at[pl.ds(0, CH[b])], ridx.at[pl.ds(b * 128, CH[b])], isem.at[b]
        ).wait()

    def start_gather(b):
        for g in range(CH[b] // LANES):
            sl = pl.ds(b * 128 + g * LANES, LANES)
            jdx[sl] = lax.shift_right_logical(ridx[sl], 1)
        if CH[b] % LANES:
            sl = pl.ds(b * 128 + CH[b] - LANES, LANES)
            jdx[sl] = lax.shift_right_logical(ridx[sl], 1)
        pltpu.async_copy(
            pair_hbm.at[jdx.at[pl.ds(b * 128, CH[b])]], bufs[b], gsem.at[b]
        )

    def wait_gather(b):
        pltpu.make_async_copy(
            pair_hbm.at[jdx.at[pl.ds(0, CH[b])]], bufs[b], gsem.at[b]
        ).wait()

    def start_store(c, b):
        seq = lax.shift_right_logical(c, 1)
        pltpu.async_copy(
            obufs[b],
            out_hbm.at[seq_base + seq, pl.ds(SOFF[b], CH[b])],
            ssem.at[b],
        )

    def wait_store(b):
        pltpu.make_async_copy(
            obufs[b], out_hbm.at[seq_base, pl.ds(SOFF[b], CH[b])], ssem.at[b]
        ).wait()

    def add_pos(b):
        buf = bufs[b]
        obuf = obufs[b]

        def do_row(i, j, srow_base, half):
            srow = srow_base + j // 2
            scol = (j & 1) * D_MODEL
            for k in range(D_MODEL // LANES):
                obuf[i, pl.ds(k * LANES, LANES)] = (
                    buf[i, pl.ds(half + k * LANES, LANES)]
                    + pos_v[srow, pl.ds(scol + k * LANES, LANES)]
                )

        def group(g, carry):
            base = g * LANES
            srow_base = SOFF[b] // 2 + g * (LANES // 2)
            hv = (ridx[pl.ds(b * 128 + base, LANES)] & 1) * D_MODEL
            for j in range(LANES):
                do_row(base + j, j, srow_base, hv[j])
            return carry

        lax.fori_loop(0, CH[b] // LANES, group, 0)
        if CH[b] % LANES:
            rem = CH[b] % LANES
            base = CH[b] - LANES
            srow_base = (SOFF[b] + base) // 2
            hv = (ridx[pl.ds(b * 128 + base, LANES)] & 1) * D_MODEL
            for j in range(LANES - rem, LANES):
                do_row(base + j, j, srow_base, hv[j])

    start_idload(0, 0)
    wait_idload(0)
    start_gather(0)
    start_idload(1, 1)

    def outer(go, carry):
        for b in range(2):
            c = go * 2 + b
            nb = 1 - b

            @pl.when(c + 1 < NCH)
            def _issue():
                @pl.when(c >= 1)
                def _drain():
                    wait_store(nb)

                wait_idload(nb)
                start_gather(nb)

            wait_gather(b)
            add_pos(b)

            @pl.when(c + 2 < NCH)
            def _kick():
                start_idload(c + 2, b)

            start_store(c, b)
        return carry

    lax.fori_loop(0, NCH // 2, outer, 0)

    for b in range(2):
        wait_store(b)


@jax.jit
def kernel(x, token_table, pos_table):
    pairs = token_table.reshape(VOCAB_SZ // 2, PAD_W)
    pos2 = pos_table.reshape(SEQ_LEN // 2, PAD_W)
    return _emb_kernel(x.reshape(-1).astype(jnp.int32), pairs, pos2)


# speedup vs baseline: 1.1771x; 1.1771x over previous
import functools

import jax
import jax.numpy as jnp
from jax import lax
from jax.experimental import pallas as pl
from jax.experimental.pallas import tpu as pltpu
from jax.experimental.pallas import tpu_sc as plsc

BATCH = 4096
SEQ_LEN = 200
D_MODEL = 64
LANES = 16

NUM_CORES = 2
NUM_SUBCORES = 16
NUM_WORKERS = NUM_CORES * NUM_SUBCORES
SEQ_PER_WORKER = BATCH // NUM_WORKERS
NBUF = 6
LEAD = 3

_mesh = plsc.VectorSubcoreMesh(core_axis_name="c", subcore_axis_name="s")


@functools.partial(
    pl.kernel,
    mesh=_mesh,
    compiler_params=pltpu.CompilerParams(use_tc_tiling_on_sc=False),
    out_type=jax.ShapeDtypeStruct((BATCH, SEQ_LEN, D_MODEL), jnp.float32),
    scratch_types=[
        pltpu.VMEM((SEQ_PER_WORKER, SEQ_LEN), jnp.int32),
        pltpu.VMEM((SEQ_LEN, D_MODEL), jnp.float32),
        pltpu.VMEM((NBUF, SEQ_LEN, D_MODEL), jnp.float32),
        pltpu.SemaphoreType.DMA((NBUF,)),
        pltpu.SemaphoreType.DMA((NBUF,)),
    ],
)
def _emb_kernel(x_hbm, tok_hbm, pos_hbm, out_hbm, idx_v, pos_v, bufs, gsem, ssem):
    cid = lax.axis_index("c")
    sid = lax.axis_index("s")
    wid = sid * NUM_CORES + cid
    seq_base = wid * SEQ_PER_WORKER

    pltpu.sync_copy(x_hbm.at[pl.ds(seq_base, SEQ_PER_WORKER)], idx_v)
    pltpu.sync_copy(pos_hbm, pos_v)

    def start_gather(c, b):
        pltpu.async_copy(tok_hbm.at[idx_v.at[c]], bufs.at[b], gsem.at[b])

    def wait_gather(b):
        pltpu.make_async_copy(tok_hbm.at[idx_v.at[0]], bufs.at[b], gsem.at[b]).wait()

    def start_store(c, b):
        pltpu.async_copy(bufs.at[b], out_hbm.at[seq_base + c], ssem.at[b])

    def wait_store(b):
        pltpu.make_async_copy(bufs.at[b], out_hbm.at[seq_base], ssem.at[b]).wait()

    def add_pos(b):
        buf = bufs.at[b]

        def row(i, carry):
            for k in range(D_MODEL // LANES):
                sl = pl.ds(k * LANES, LANES)
                buf[i, sl] = buf[i, sl] + pos_v[i, sl]
            return carry

        lax.fori_loop(0, SEQ_LEN, row, 0, unroll=2)

    for b in range(LEAD):
        start_gather(b, b)

    def outer(go, carry):
        for b in range(NBUF):
            c = go * NBUF + b
            nslot = (b + LEAD) % NBUF

            @pl.when(c < SEQ_PER_WORKER - LEAD)
            def _issue():
                @pl.when(c >= NBUF - LEAD)
                def _drain():
                    wait_store(nslot)

                start_gather(c + LEAD, nslot)

            wait_gather(b)
            add_pos(b)
            start_store(c, b)
        return carry

    main = SEQ_PER_WORKER - (SEQ_PER_WORKER % NBUF)
    lax.fori_loop(0, main // NBUF, outer, 0)
    for c in range(main, SEQ_PER_WORKER):
        b = c % NBUF
        wait_gather(b)
        add_pos(b)
        start_store(c, b)

    for b in range(NBUF):
        wait_store(b)


@jax.jit
def kernel(x, token_table, pos_table):
    return _emb_kernel(x.astype(jnp.int32), token_table, pos_table)


# speedup vs baseline: 1.1894x; 1.0105x over previous
import functools

import jax
import jax.numpy as jnp
from jax import lax
from jax.experimental import pallas as pl
from jax.experimental.pallas import tpu as pltpu
from jax.experimental.pallas import tpu_sc as plsc

BATCH = 4096
SEQ_LEN = 200
D_MODEL = 64
LANES = 16

NUM_CORES = 2
NUM_SUBCORES = 16
NUM_WORKERS = NUM_CORES * NUM_SUBCORES
SEQ_PER_WORKER = BATCH // NUM_WORKERS
NBUF = 7
LEAD = 4

_mesh = plsc.VectorSubcoreMesh(core_axis_name="c", subcore_axis_name="s")


@functools.partial(
    pl.kernel,
    mesh=_mesh,
    compiler_params=pltpu.CompilerParams(use_tc_tiling_on_sc=False),
    out_type=jax.ShapeDtypeStruct((BATCH, SEQ_LEN, D_MODEL), jnp.float32),
    scratch_types=[
        pltpu.VMEM((SEQ_PER_WORKER, SEQ_LEN), jnp.int32),
        pltpu.VMEM((SEQ_LEN, D_MODEL), jnp.float32),
        pltpu.VMEM((NBUF, SEQ_LEN, D_MODEL), jnp.float32),
        pltpu.SemaphoreType.DMA((NBUF,)),
        pltpu.SemaphoreType.DMA((NBUF,)),
    ],
)
def _emb_kernel(x_hbm, tok_hbm, pos_hbm, out_hbm, idx_v, pos_v, bufs, gsem, ssem):
    cid = lax.axis_index("c")
    sid = lax.axis_index("s")
    wid = sid * NUM_CORES + cid
    seq_base = wid * SEQ_PER_WORKER

    pltpu.sync_copy(x_hbm.at[pl.ds(seq_base, SEQ_PER_WORKER)], idx_v)
    pltpu.sync_copy(pos_hbm, pos_v)

    def start_gather(c, b):
        pltpu.async_copy(tok_hbm.at[idx_v.at[c]], bufs.at[b], gsem.at[b])

    def wait_gather(b):
        pltpu.make_async_copy(tok_hbm.at[idx_v.at[0]], bufs.at[b], gsem.at[b]).wait()

    def start_store(c, b):
        pltpu.async_copy(bufs.at[b], out_hbm.at[seq_base + c], ssem.at[b])

    def wait_store(b):
        pltpu.make_async_copy(bufs.at[b], out_hbm.at[seq_base], ssem.at[b]).wait()

    def add_pos(b):
        buf = bufs.at[b]

        def row(i, carry):
            for k in range(D_MODEL // LANES):
                sl = pl.ds(k * LANES, LANES)
                buf[i, sl] = buf[i, sl] + pos_v[i, sl]
            return carry

        lax.fori_loop(0, SEQ_LEN, row, 0, unroll=4)

    for b in range(LEAD):
        start_gather(b, b)

    def outer(go, carry):
        for b in range(NBUF):
            c = go * NBUF + b
            nslot = (b + LEAD) % NBUF

            @pl.when(c < SEQ_PER_WORKER - LEAD)
            def _issue():
                @pl.when(c >= NBUF - LEAD)
                def _drain():
                    wait_store(nslot)

                start_gather(c + LEAD, nslot)

            wait_gather(b)
            add_pos(b)
            start_store(c, b)
        return carry

    main = SEQ_PER_WORKER - (SEQ_PER_WORKER % NBUF)
    lax.fori_loop(0, main // NBUF, outer, 0)
    for c in range(main, SEQ_PER_WORKER):
        b = c % NBUF
        wait_gather(b)
        add_pos(b)
        start_store(c, b)

    for b in range(NBUF):
        wait_store(b)


@jax.jit
def kernel(x, token_table, pos_table):
    return _emb_kernel(x.astype(jnp.int32), token_table, pos_table)
